# Initial kernel scaffold; baseline (speedup 1.0000x reference)
import functools

import jax
import jax.numpy as jnp
from jax import lax
from jax.experimental import pallas as pl
from jax.experimental.pallas import tpu as pltpu
from jax.experimental.pallas import tpu_sc as plsc

N_NODES_P = 10016
ROWS_PER_TILE = 626
E_PAD = 323584
CHUNKS = 79
CHUNK = 128
NC, NS = 2, 16
PAD_NODE = 10008
F = 64
DEG_W = 16


def _sc_mesh():
    return plsc.VectorSubcoreMesh(core_axis_name="c", subcore_axis_name="s",
                                  num_cores=NC, num_subcores=NS)


def _make_degree_kernel():
    mesh = _sc_mesh()

    @functools.partial(
        pl.kernel,
        out_type=jax.ShapeDtypeStruct((NC, N_NODES_P, DEG_W), jnp.float32),
        mesh=mesh,
        scratch_types=[
            pltpu.VMEM((CHUNKS, CHUNK), jnp.int32),
            pltpu.VMEM((CHUNK, DEG_W), jnp.float32),
            pltpu.VMEM((ROWS_PER_TILE, DEG_W), jnp.float32),
            pltpu.VMEM_SHARED((N_NODES_P, DEG_W), jnp.float32),
            pltpu.SemaphoreType.DMA,
        ],
    )
    def deg_kernel(dst_hbm, out_hbm, dst_v, ones_v, zbuf, acc_sh, sem):
        cid = lax.axis_index("c")
        sid = lax.axis_index("s")
        wid = sid * NC + cid

        one_row = jnp.where(lax.iota(jnp.int32, 16) == 0, 1.0, 0.0).astype(jnp.float32)
        zero = jnp.zeros((16,), jnp.float32)

        def fill(i, _):
            ones_v[i, :] = one_row
            return 0
        lax.fori_loop(0, CHUNK, fill, 0)

        def zfill(i, _):
            zbuf[i, :] = zero
            return 0
        lax.fori_loop(0, ROWS_PER_TILE, zfill, 0)

        pltpu.sync_copy(zbuf, acc_sh.at[pl.ds(sid * ROWS_PER_TILE, ROWS_PER_TILE)])
        plsc.subcore_barrier()

        pltpu.sync_copy(dst_hbm.at[wid], dst_v)

        def chunk(j, _):
            pltpu.sync_copy(ones_v, acc_sh.at[dst_v.at[j]], add=True)
            return 0
        lax.fori_loop(0, CHUNKS, chunk, 0)

        plsc.subcore_barrier()
        pltpu.sync_copy(
            acc_sh.at[pl.ds(sid * ROWS_PER_TILE, ROWS_PER_TILE)],
            out_hbm.at[cid, pl.ds(sid * ROWS_PER_TILE, ROWS_PER_TILE)])

    return deg_kernel


def _make_aggregate_kernel():
    mesh = _sc_mesh()

    @functools.partial(
        pl.kernel,
        out_type=jax.ShapeDtypeStruct((NC, N_NODES_P, F), jnp.float32),
        mesh=mesh,
        scratch_types=[
            pltpu.VMEM((CHUNKS, CHUNK), jnp.int32),
            pltpu.VMEM((CHUNKS, CHUNK), jnp.int32),
            pltpu.VMEM((CHUNK, F), jnp.float32),
            pltpu.VMEM((ROWS_PER_TILE, F), jnp.float32),
            pltpu.VMEM_SHARED((N_NODES_P, F), jnp.float32),
            pltpu.SemaphoreType.DMA,
        ],
    )
    def agg_kernel(src_hbm, dst_hbm, y_hbm, out_hbm,
                   src_v, dst_v, rows_v, zbuf, acc_sh, sem):
        cid = lax.axis_index("c")
        sid = lax.axis_index("s")
        wid = sid * NC + cid

        zero = jnp.zeros((16,), jnp.float32)

        def zfill(i, _):
            for j in range(F // 16):
                zbuf[i, pl.ds(j * 16, 16)] = zero
            return 0
        lax.fori_loop(0, ROWS_PER_TILE, zfill, 0)

        pltpu.sync_copy(zbuf, acc_sh.at[pl.ds(sid * ROWS_PER_TILE, ROWS_PER_TILE)])
        plsc.subcore_barrier()

        pltpu.sync_copy(src_hbm.at[wid], src_v)
        pltpu.sync_copy(dst_hbm.at[wid], dst_v)

        def chunk(j, _):
            pltpu.async_copy(y_hbm.at[src_v.at[j]], rows_v, sem).wait()
            pltpu.sync_copy(rows_v, acc_sh.at[dst_v.at[j]], add=True)
            return 0
        lax.fori_loop(0, CHUNKS, chunk, 0)

        plsc.subcore_barrier()
        pltpu.sync_copy(
            acc_sh.at[pl.ds(sid * ROWS_PER_TILE, ROWS_PER_TILE)],
            out_hbm.at[cid, pl.ds(sid * ROWS_PER_TILE, ROWS_PER_TILE)])

    return agg_kernel


def _mm1_body(x_ref, w_ref, dis_ref, y_ref):
    xw = jnp.dot(x_ref[...], w_ref[...], preferred_element_type=jnp.float32)
    y_ref[...] = dis_ref[...] * xw


def _tc_mm1(x_pad, W1, dis):
    return pl.pallas_call(
        _mm1_body,
        out_shape=jax.ShapeDtypeStruct((N_NODES_P, F), jnp.float32),
    )(x_pad, W1, dis)


def _mid_body(agg_ref, dis_ref, b_ref, w_ref, xk_ref, y_ref):
    agg = agg_ref[0] + agg_ref[1]
    dis = dis_ref[...]
    xk = jnp.maximum(dis * agg + b_ref[...], 0.0)
    xk_ref[...] = xk
    y_ref[...] = dis * jnp.dot(xk, w_ref[...], preferred_element_type=jnp.float32)


def _tc_mid(agg_p, dis, b, W_next):
    return pl.pallas_call(
        _mid_body,
        out_shape=[jax.ShapeDtypeStruct((N_NODES_P, F), jnp.float32),
                   jax.ShapeDtypeStruct((N_NODES_P, F), jnp.float32)],
    )(agg_p, dis, b, W_next)


def _final_body(agg_ref, dis_ref, b_ref, x1_ref, x2_ref, batch_ref,
                wf_ref, bf_ref, out_ref):
    agg = agg_ref[0] + agg_ref[1]
    x3 = jnp.maximum(dis_ref[...] * agg + b_ref[...], 0.0)

    gids = lax.broadcasted_iota(jnp.int32, (1, 64), 1)
    P = (batch_ref[...] == gids).astype(jnp.float32)

    dn = (((0,), (0,)), ((), ()))
    s1 = lax.dot_general(P, x1_ref[...], dn, preferred_element_type=jnp.float32)
    s2 = lax.dot_general(P, x2_ref[...], dn, preferred_element_type=jnp.float32)
    s3 = lax.dot_general(P, x3, dn, preferred_element_type=jnp.float32)
    pooled = jnp.concatenate([s1, s2, s3], axis=1)

    counts = jnp.sum(P, axis=0, keepdims=True)
    inv = 1.0 / jnp.maximum(counts, 1.0)
    pooled = pooled * inv.T

    logits = jnp.dot(pooled, wf_ref[...], preferred_element_type=jnp.float32)
    logits = logits + bf_ref[...]
    m = jnp.max(logits, axis=1, keepdims=True)
    e = jnp.exp(logits - m)
    out_ref[...] = e / jnp.sum(e, axis=1, keepdims=True)


def _tc_final(agg_p, dis, b3, x1, x2, batch_pad, Wf, bf):
    return pl.pallas_call(
        _final_body,
        out_shape=jax.ShapeDtypeStruct((64, 10), jnp.float32),
    )(agg_p, dis, b3, x1, x2, batch_pad, Wf, bf)


def kernel(x, edge_index, batch, W1, b1, W2, b2, W3, b3, Wf, bf):
    n = x.shape[0]
    src = edge_index[0].astype(jnp.int32)
    dst = edge_index[1].astype(jnp.int32)

    pad_e = E_PAD - src.shape[0]
    pad_idx = jnp.full((pad_e,), PAD_NODE, jnp.int32)
    src3 = jnp.concatenate([src, pad_idx]).reshape(NC * NS, CHUNKS, CHUNK)
    dst3 = jnp.concatenate([dst, pad_idx]).reshape(NC * NS, CHUNKS, CHUNK)

    x_pad = jnp.concatenate(
        [x, jnp.zeros((N_NODES_P - n, x.shape[1]), x.dtype)], axis=0)
    batch_pad = jnp.concatenate(
        [batch.astype(jnp.int32), jnp.full((N_NODES_P - n,), 64, jnp.int32)]
    ).reshape(N_NODES_P, 1)

    deg_kernel = _make_degree_kernel()
    agg_kernel = _make_aggregate_kernel()

    deg_parts = deg_kernel(dst3)
    deg = deg_parts[0, :, 0] + deg_parts[1, :, 0]
    dis = jnp.where(deg > 0, lax.rsqrt(jnp.maximum(deg, 1e-30)), 0.0)
    dis = dis.reshape(N_NODES_P, 1)

    b1r = b1.reshape(1, F)
    b2r = b2.reshape(1, F)
    b3r = b3.reshape(1, F)
    bfr = bf.reshape(1, 10)

    y1 = _tc_mm1(x_pad, W1, dis)
    agg1 = agg_kernel(src3, dst3, y1)
    x1, y2 = _tc_mid(agg1, dis, b1r, W2)
    agg2 = agg_kernel(src3, dst3, y2)
    x2, y3 = _tc_mid(agg2, dis, b2r, W3)
    agg3 = agg_kernel(src3, dst3, y3)
    return _tc_final(agg3, dis, b3r, x1, x2, batch_pad, Wf, bfr)


# speedup vs baseline: 13.3678x; 13.3678x over previous
import functools

import jax
import jax.numpy as jnp
from jax import lax
from jax.experimental import pallas as pl
from jax.experimental.pallas import tpu as pltpu
from jax.experimental.pallas import tpu_sc as plsc

N_NODES_P = 10240
ROWS_PER_TILE = 640
E_PAD = 323584
CHUNKS = 79
CHUNK = 128
NC, NS = 2, 16
PAD_NODE = 10008
F = 64
DEG_W = 16


def _sc_mesh():
    return plsc.VectorSubcoreMesh(core_axis_name="c", subcore_axis_name="s",
                                  num_cores=NC, num_subcores=NS)


def _make_degree_kernel():
    mesh = _sc_mesh()

    @functools.partial(
        pl.kernel,
        out_type=jax.ShapeDtypeStruct((NC, N_NODES_P, DEG_W), jnp.float32),
        mesh=mesh,
        compiler_params=pltpu.CompilerParams(use_tc_tiling_on_sc=False),
        scratch_types=[
            pltpu.VMEM((CHUNKS, CHUNK), jnp.int32),
            pltpu.VMEM((CHUNK, DEG_W), jnp.float32),
            pltpu.VMEM((ROWS_PER_TILE, DEG_W), jnp.float32),
            pltpu.VMEM_SHARED((N_NODES_P, DEG_W), jnp.float32),
            pltpu.SemaphoreType.DMA,
        ],
    )
    def deg_kernel(dst_hbm, out_hbm, dst_v, ones_v, zbuf, acc_sh, sem):
        cid = lax.axis_index("c")
        sid = lax.axis_index("s")
        wid = sid * NC + cid

        one_row = jnp.where(lax.iota(jnp.int32, 16) == 0, 1.0, 0.0).astype(jnp.float32)
        zero = jnp.zeros((16,), jnp.float32)

        def fill(i, _):
            ones_v[i, :] = one_row
            return 0
        lax.fori_loop(0, CHUNK, fill, 0)

        def zfill(i, _):
            zbuf[i, :] = zero
            return 0
        lax.fori_loop(0, ROWS_PER_TILE, zfill, 0)

        pltpu.sync_copy(zbuf, acc_sh.at[pl.ds(sid * ROWS_PER_TILE, ROWS_PER_TILE)])
        plsc.subcore_barrier()

        pltpu.sync_copy(dst_hbm.at[wid], dst_v)

        def chunk(j, _):
            pltpu.sync_copy(ones_v, acc_sh.at[dst_v.at[j]], add=True)
            return 0
        lax.fori_loop(0, CHUNKS, chunk, 0)

        plsc.subcore_barrier()
        pltpu.sync_copy(
            acc_sh.at[pl.ds(sid * ROWS_PER_TILE, ROWS_PER_TILE)],
            out_hbm.at[cid, pl.ds(sid * ROWS_PER_TILE, ROWS_PER_TILE)])

    return deg_kernel


def _make_aggregate_kernel():
    mesh = _sc_mesh()

    @functools.partial(
        pl.kernel,
        out_type=jax.ShapeDtypeStruct((NC, N_NODES_P, F), jnp.float32),
        mesh=mesh,
        compiler_params=pltpu.CompilerParams(use_tc_tiling_on_sc=False),
        scratch_types=[
            pltpu.VMEM((CHUNKS, CHUNK), jnp.int32),
            pltpu.VMEM((CHUNKS, CHUNK), jnp.int32),
            pltpu.VMEM((CHUNK, F), jnp.float32),
            pltpu.VMEM((ROWS_PER_TILE, F), jnp.float32),
            pltpu.VMEM_SHARED((N_NODES_P, F), jnp.float32),
            pltpu.SemaphoreType.DMA,
        ],
    )
    def agg_kernel(src_hbm, dst_hbm, y_hbm, out_hbm,
                   src_v, dst_v, rows_v, zbuf, acc_sh, sem):
        cid = lax.axis_index("c")
        sid = lax.axis_index("s")
        wid = sid * NC + cid

        zero = jnp.zeros((16,), jnp.float32)

        def zfill(i, _):
            for j in range(F // 16):
                zbuf[i, pl.ds(j * 16, 16)] = zero
            return 0
        lax.fori_loop(0, ROWS_PER_TILE, zfill, 0)

        pltpu.sync_copy(zbuf, acc_sh.at[pl.ds(sid * ROWS_PER_TILE, ROWS_PER_TILE)])
        plsc.subcore_barrier()

        pltpu.sync_copy(src_hbm.at[wid], src_v)
        pltpu.sync_copy(dst_hbm.at[wid], dst_v)

        def chunk(j, _):
            pltpu.async_copy(y_hbm.at[src_v.at[j]], rows_v, sem).wait()
            pltpu.sync_copy(rows_v, acc_sh.at[dst_v.at[j]], add=True)
            return 0
        lax.fori_loop(0, CHUNKS, chunk, 0)

        plsc.subcore_barrier()
        pltpu.sync_copy(
            acc_sh.at[pl.ds(sid * ROWS_PER_TILE, ROWS_PER_TILE)],
            out_hbm.at[cid, pl.ds(sid * ROWS_PER_TILE, ROWS_PER_TILE)])

    return agg_kernel


def _mm1_body(x_ref, w_ref, dis_ref, y_ref):
    xw = jnp.dot(x_ref[...], w_ref[...], preferred_element_type=jnp.float32)
    y_ref[...] = dis_ref[...] * xw


def _tc_mm1(x_pad, W1, dis):
    return pl.pallas_call(
        _mm1_body,
        out_shape=jax.ShapeDtypeStruct((N_NODES_P, F), jnp.float32),
    )(x_pad, W1, dis)


def _mid_body(agg_ref, dis_ref, b_ref, w_ref, xk_ref, y_ref):
    agg = agg_ref[0] + agg_ref[1]
    dis = dis_ref[...]
    xk = jnp.maximum(dis * agg + b_ref[...], 0.0)
    xk_ref[...] = xk
    y_ref[...] = dis * jnp.dot(xk, w_ref[...], preferred_element_type=jnp.float32)


def _tc_mid(agg_p, dis, b, W_next):
    return pl.pallas_call(
        _mid_body,
        out_shape=[jax.ShapeDtypeStruct((N_NODES_P, F), jnp.float32),
                   jax.ShapeDtypeStruct((N_NODES_P, F), jnp.float32)],
    )(agg_p, dis, b, W_next)


def _final_body(agg_ref, dis_ref, b_ref, x1_ref, x2_ref, batch_ref,
                wf_ref, bf_ref, out_ref):
    agg = agg_ref[0] + agg_ref[1]
    x3 = jnp.maximum(dis_ref[...] * agg + b_ref[...], 0.0)

    gids = lax.broadcasted_iota(jnp.int32, (1, 64), 1)
    P = (batch_ref[...] == gids).astype(jnp.float32)

    dn = (((0,), (0,)), ((), ()))
    s1 = lax.dot_general(P, x1_ref[...], dn, preferred_element_type=jnp.float32)
    s2 = lax.dot_general(P, x2_ref[...], dn, preferred_element_type=jnp.float32)
    s3 = lax.dot_general(P, x3, dn, preferred_element_type=jnp.float32)
    pooled = jnp.concatenate([s1, s2, s3], axis=1)

    counts = jnp.sum(P, axis=0, keepdims=True)
    inv = 1.0 / jnp.maximum(counts, 1.0)
    pooled = pooled * inv.T

    logits = jnp.dot(pooled, wf_ref[...], preferred_element_type=jnp.float32)
    logits = logits + bf_ref[...]
    m = jnp.max(logits, axis=1, keepdims=True)
    e = jnp.exp(logits - m)
    out_ref[...] = e / jnp.sum(e, axis=1, keepdims=True)


def _tc_final(agg_p, dis, b3, x1, x2, batch_pad, Wf, bf):
    return pl.pallas_call(
        _final_body,
        out_shape=jax.ShapeDtypeStruct((64, 10), jnp.float32),
    )(agg_p, dis, b3, x1, x2, batch_pad, Wf, bf)


def kernel(x, edge_index, batch, W1, b1, W2, b2, W3, b3, Wf, bf):
    n = x.shape[0]
    src = edge_index[0].astype(jnp.int32)
    dst = edge_index[1].astype(jnp.int32)

    pad_e = E_PAD - src.shape[0]
    pad_idx = jnp.full((pad_e,), PAD_NODE, jnp.int32)
    src3 = jnp.concatenate([src, pad_idx]).reshape(NC * NS, CHUNKS, CHUNK)
    dst3 = jnp.concatenate([dst, pad_idx]).reshape(NC * NS, CHUNKS, CHUNK)

    x_pad = jnp.concatenate(
        [x, jnp.zeros((N_NODES_P - n, x.shape[1]), x.dtype)], axis=0)
    batch_pad = jnp.concatenate(
        [batch.astype(jnp.int32), jnp.full((N_NODES_P - n,), 64, jnp.int32)]
    ).reshape(N_NODES_P, 1)

    deg_kernel = _make_degree_kernel()
    agg_kernel = _make_aggregate_kernel()

    deg_parts = deg_kernel(dst3)
    deg = deg_parts[0, :, 0] + deg_parts[1, :, 0]
    dis = jnp.where(deg > 0, lax.rsqrt(jnp.maximum(deg, 1e-30)), 0.0)
    dis = dis.reshape(N_NODES_P, 1)

    b1r = b1.reshape(1, F)
    b2r = b2.reshape(1, F)
    b3r = b3.reshape(1, F)
    bfr = bf.reshape(1, 10)

    y1 = _tc_mm1(x_pad, W1, dis)
    agg1 = agg_kernel(src3, dst3, y1)
    x1, y2 = _tc_mid(agg1, dis, b1r, W2)
    agg2 = agg_kernel(src3, dst3, y2)
    x2, y3 = _tc_mid(agg2, dis, b2r, W3)
    agg3 = agg_kernel(src3, dst3, y3)
    return _tc_final(agg3, dis, b3r, x1, x2, batch_pad, Wf, bfr)
